# Initial kernel scaffold; baseline (speedup 1.0000x reference)
import jax
import jax.numpy as jnp
from jax.experimental import pallas as pl


def kernel(nodes, history_uv, history_r, v2e_weight, u2e_weight, r2e_weight, W1, b1, W2, b2):
    raise NotImplementedError("write your pallas kernel here")


# speedup vs baseline: 16.4199x; 16.4199x over previous
import functools

import numpy as np
import jax
import jax.numpy as jnp
from jax import lax
from jax.experimental import pallas as pl
from jax.experimental.pallas import tpu as pltpu
from jax.experimental.pallas import tpu_sc as plsc

_NC, _NS = 2, 16
_NW = _NC * _NS

_CHUNK = 256
_NBUF = 4

_BB = 128


def _sc_gather(table, idx_flat):
    n = idx_flat.shape[0]
    _, d = table.shape
    per_w = n // _NW
    nchunks = per_w // _CHUNK
    nloops = nchunks // _NBUF
    assert per_w * _NW == n and _CHUNK * nchunks == per_w and _NBUF * nloops == nchunks

    mesh = plsc.VectorSubcoreMesh(
        core_axis_name="c", subcore_axis_name="s", num_cores=_NC, num_subcores=_NS
    )

    @functools.partial(
        pl.kernel,
        out_type=jax.ShapeDtypeStruct((n, d), jnp.float32),
        mesh=mesh,
        scratch_types=[
            pltpu.VMEM((per_w,), jnp.int32),
            *[pltpu.VMEM((_CHUNK, d), jnp.float32) for _ in range(_NBUF)],
            *[pltpu.SemaphoreType.DMA for _ in range(2 * _NBUF)],
        ],
        compiler_params=pltpu.CompilerParams(use_tc_tiling_on_sc=False),
    )
    def gather_kernel(table_hbm, idx_hbm, out_hbm, idx_v, *rest):
        rows = rest[:_NBUF]
        gsems = rest[_NBUF : 2 * _NBUF]
        wsems = rest[2 * _NBUF :]
        wid = lax.axis_index("s") * _NC + lax.axis_index("c")
        base = wid * per_w
        pltpu.sync_copy(idx_hbm.at[pl.ds(base, per_w)], idx_v)

        def g_start(c, b):
            pltpu.async_copy(
                table_hbm.at[idx_v.at[pl.ds(c * _CHUNK, _CHUNK)]], rows[b], gsems[b]
            )

        def g_wait(b):
            pltpu.make_async_copy(
                table_hbm.at[idx_v.at[pl.ds(0, _CHUNK)]], rows[b], gsems[b]
            ).wait()

        def w_start(c, b):
            pltpu.async_copy(
                rows[b], out_hbm.at[pl.ds(base + c * _CHUNK, _CHUNK)], wsems[b]
            )

        def w_wait(b):
            pltpu.make_async_copy(
                rows[b], out_hbm.at[pl.ds(0, _CHUNK)], wsems[b]
            ).wait()

        for b in range(_NBUF):
            g_start(b, b)

        def body(i, carry):
            for b in range(_NBUF):
                g_wait(b)
                w_start(i * _NBUF + b, b)

            @pl.when(i < nloops - 1)
            def _prefetch():
                for b in range(_NBUF):
                    w_wait(b)
                    g_start(i * _NBUF + b + _NBUF, b)

            return carry

        lax.fori_loop(0, nloops, body, 0)
        for b in range(_NBUF):
            w_wait(b)

    return gather_kernel(table, idx_flat)


def _pair_selectors(r):
    sel_e = np.zeros((32, 8), np.float32)
    sel_o = np.zeros((32, 8), np.float32)
    for p in range(r * r):
        sel_e[p, p // r] = 1.0
        sel_o[p, p % r] = 1.0
    return sel_e, sel_o


def _tc_mlp_body(
    g_ref, rp_ref, r2e_ref, w1b_ref, b1_ref, sel_e_ref, sel_o_ref, w1blk_ref,
    w2blk_ref, b2p_ref, out_ref, *, bb, ll, d, r
):
    bbl2 = bb * ll // 2
    rtab = (
        jnp.dot(r2e_ref[...], w1b_ref[...], preferred_element_type=jnp.float32)
        + b1_ref[...]
    )
    rtab_pair = jnp.concatenate(
        [
            jnp.dot(sel_e_ref[...], rtab, preferred_element_type=jnp.float32),
            jnp.dot(sel_o_ref[...], rtab, preferred_element_type=jnp.float32),
        ],
        axis=1,
    )
    rp_row = rp_ref[0]
    ohp = jnp.equal(
        lax.broadcasted_iota(jnp.int32, (32, bbl2), 0), rp_row
    ).astype(jnp.float32)
    radd = lax.dot_general(
        ohp, rtab_pair, (((0,), (0,)), ((), ())), preferred_element_type=jnp.float32
    )
    x = g_ref[...]
    h = jnp.maximum(
        jnp.dot(x, w1blk_ref[...], preferred_element_type=jnp.float32) + radd, 0.0
    )
    o = jnp.maximum(
        jnp.dot(h, w2blk_ref[...], preferred_element_type=jnp.float32) + b2p_ref[...],
        0.0,
    )
    s = o.reshape(bb, ll // 2, 2 * d).sum(axis=1)
    out_ref[...] = (s[:, :d] + s[:, d:]) * (1.0 / ll)


def _tc_mlp(g2, rp, r2e8, w1b, b1, sel_e, sel_o, w1blk, w2blk, b2p, *, b_total, ll, d, r):
    nblocks = b_total // _BB
    bbl2 = _BB * ll // 2
    body = functools.partial(_tc_mlp_body, bb=_BB, ll=ll, d=d, r=r)
    return pl.pallas_call(
        body,
        grid=(nblocks,),
        in_specs=[
            pl.BlockSpec((bbl2, 2 * d), lambda i: (i, 0)),
            pl.BlockSpec((1, 1, bbl2), lambda i: (i, 0, 0)),
            pl.BlockSpec((8, d), lambda i: (0, 0)),
            pl.BlockSpec((d, d), lambda i: (0, 0)),
            pl.BlockSpec((1, d), lambda i: (0, 0)),
            pl.BlockSpec((32, 8), lambda i: (0, 0)),
            pl.BlockSpec((32, 8), lambda i: (0, 0)),
            pl.BlockSpec((2 * d, 2 * d), lambda i: (0, 0)),
            pl.BlockSpec((2 * d, 2 * d), lambda i: (0, 0)),
            pl.BlockSpec((1, 2 * d), lambda i: (0, 0)),
        ],
        out_specs=pl.BlockSpec((_BB, d), lambda i: (i, 0)),
        out_shape=jax.ShapeDtypeStruct((b_total, d), jnp.float32),
    )(g2, rp, r2e8, w1b, b1, sel_e, sel_o, w1blk, w2blk, b2p)


def kernel(nodes, history_uv, history_r, v2e_weight, u2e_weight, r2e_weight, W1, b1, W2, b2):
    b_total, ll = history_uv.shape
    _, d = v2e_weight.shape
    r = r2e_weight.shape[0]
    n = b_total * ll

    idx_flat = history_uv.reshape(-1).astype(jnp.int32)
    g_flat = _sc_gather(v2e_weight, idx_flat)
    g2 = g_flat.reshape(n // 2, 2 * d)

    r_flat = history_r.astype(jnp.int32).reshape(-1)
    rp = r_flat[0::2] * r + r_flat[1::2]
    rp = rp.reshape(b_total // _BB, 1, _BB * ll // 2)

    r2e8 = jnp.pad(r2e_weight, ((0, 8 - r), (0, 0)))
    w1a = W1[:d, :]
    w1b = W1[d:, :]
    zero = jnp.zeros((d, d), jnp.float32)
    w1blk = jnp.block([[w1a, zero], [zero, w1a]])
    w2blk = jnp.block([[W2, zero], [zero, W2]])
    b2p = jnp.concatenate([b2, b2]).reshape(1, 2 * d)
    sel_e, sel_o = _pair_selectors(r)

    return _tc_mlp(
        g2, rp, r2e8, w1b, b1.reshape(1, d), jnp.asarray(sel_e), jnp.asarray(sel_o),
        w1blk, w2blk, b2p, b_total=b_total, ll=ll, d=d, r=r,
    )


# speedup vs baseline: 18.8228x; 1.1463x over previous
import functools

import numpy as np
import jax
import jax.numpy as jnp
from jax import lax
from jax.experimental import pallas as pl
from jax.experimental.pallas import tpu as pltpu
from jax.experimental.pallas import tpu_sc as plsc

_NC, _NS = 2, 16
_NW = _NC * _NS

_CHUNK = 256
_NBUF = 4

_BB = 128


def _sc_gather(table, idx_flat):
    n = idx_flat.shape[0]
    _, d = table.shape
    per_w = n // _NW
    nchunks = per_w // _CHUNK
    nloops = nchunks // _NBUF
    assert per_w * _NW == n and _CHUNK * nchunks == per_w and _NBUF * nloops == nchunks

    mesh = plsc.VectorSubcoreMesh(
        core_axis_name="c", subcore_axis_name="s", num_cores=_NC, num_subcores=_NS
    )

    @functools.partial(
        pl.kernel,
        out_type=jax.ShapeDtypeStruct((n, d), jnp.float32),
        mesh=mesh,
        scratch_types=[
            pltpu.VMEM((per_w,), jnp.int32),
            pltpu.VMEM((per_w,), jnp.int32),
            *[pltpu.VMEM((_CHUNK, d), jnp.float32) for _ in range(_NBUF)],
            *[pltpu.SemaphoreType.DMA for _ in range(2 * _NBUF)],
        ],
        compiler_params=pltpu.CompilerParams(
            use_tc_tiling_on_sc=False, needs_layout_passes=False
        ),
    )
    def gather_kernel(table_hbm, idx_hbm, out_hbm, idx_v, idx_p, *rest):
        rows = rest[:_NBUF]
        gsems = rest[_NBUF : 2 * _NBUF]
        wsems = rest[2 * _NBUF :]
        wid = lax.axis_index("s") * _NC + lax.axis_index("c")
        base = wid * per_w
        pltpu.sync_copy(idx_hbm.at[pl.ds(base, per_w)], idx_v)

        half = per_w // 2

        def perm_body(k, carry):
            va = idx_v[pl.ds(k * 16, 16)]
            vb = idx_v[pl.ds(half + k * 16, 16)]
            lanes = k * 32 + lax.broadcasted_iota(jnp.int32, (16,), 0) * 2
            plsc.store_scatter(idx_p, [lanes], va)
            plsc.store_scatter(idx_p, [lanes + 1], vb)
            return carry

        lax.fori_loop(0, half // 16, perm_body, 0)

        def g_start(c, b):
            pltpu.async_copy(
                table_hbm.at[idx_p.at[pl.ds(c * _CHUNK, _CHUNK)]], rows[b], gsems[b]
            )

        def g_wait(b):
            pltpu.make_async_copy(
                table_hbm.at[idx_p.at[pl.ds(0, _CHUNK)]], rows[b], gsems[b]
            ).wait()

        def w_start(c, b):
            pltpu.async_copy(
                rows[b], out_hbm.at[pl.ds(base + c * _CHUNK, _CHUNK)], wsems[b]
            )

        def w_wait(b):
            pltpu.make_async_copy(
                rows[b], out_hbm.at[pl.ds(0, _CHUNK)], wsems[b]
            ).wait()

        for b in range(_NBUF):
            g_start(b, b)

        def body(i, carry):
            for b in range(_NBUF):
                g_wait(b)
                w_start(i * _NBUF + b, b)

            @pl.when(i < nloops - 1)
            def _prefetch():
                for b in range(_NBUF):
                    w_wait(b)
                    g_start(i * _NBUF + b + _NBUF, b)

            return carry

        lax.fori_loop(0, nloops, body, 0)
        for b in range(_NBUF):
            w_wait(b)

    return gather_kernel(table, idx_flat)


def _tc_mlp_body(
    g_ref, rt_ref, r2e_ref, w1b_ref, b1_ref, w1blk_ref,
    w2blk_ref, b2p_ref, out_ref, *, bb, ll, d, r
):
    bbl = bb * ll
    bbl2 = bbl // 2
    rtab = (
        jnp.dot(r2e_ref[...], w1b_ref[...], preferred_element_type=jnp.float32)
        + b1_ref[...]
    )
    zeros = jnp.zeros((8, d), jnp.float32)
    rtab16 = jnp.concatenate(
        [
            jnp.concatenate([rtab, zeros], axis=1),
            jnp.concatenate([zeros, rtab], axis=1),
        ],
        axis=0,
    )
    rt_row = rt_ref[0]
    iot = lax.broadcasted_iota(jnp.int32, (8, bbl2), 0)
    oh16 = jnp.concatenate(
        [
            jnp.equal(iot, rt_row[:, :bbl2]).astype(jnp.float32),
            jnp.equal(iot, rt_row[:, bbl2:]).astype(jnp.float32),
        ],
        axis=0,
    )
    radd = lax.dot_general(
        oh16, rtab16, (((0,), (0,)), ((), ())), preferred_element_type=jnp.float32
    )
    x = g_ref[...]
    h = jnp.maximum(
        jnp.dot(x, w1blk_ref[...], preferred_element_type=jnp.float32) + radd, 0.0
    )
    o = jnp.maximum(
        jnp.dot(h, w2blk_ref[...], preferred_element_type=jnp.float32) + b2p_ref[...],
        0.0,
    )
    s = o.reshape(bb // 2, ll, 2 * d).sum(axis=1)
    out_ref[...] = (
        jnp.concatenate([s[:, :d], s[:, d:]], axis=0) * (1.0 / ll)
    )


def _tc_mlp(g2, rt, r2e8, w1b, b1, w1blk, w2blk, b2p, *, b_total, ll, d, r):
    nblocks = b_total // _BB
    bbl = _BB * ll
    body = functools.partial(_tc_mlp_body, bb=_BB, ll=ll, d=d, r=r)
    return pl.pallas_call(
        body,
        grid=(nblocks,),
        in_specs=[
            pl.BlockSpec((bbl // 2, 2 * d), lambda i: (i, 0)),
            pl.BlockSpec((1, 1, bbl), lambda i: (i, 0, 0)),
            pl.BlockSpec((8, d), lambda i: (0, 0)),
            pl.BlockSpec((d, d), lambda i: (0, 0)),
            pl.BlockSpec((1, d), lambda i: (0, 0)),
            pl.BlockSpec((2 * d, 2 * d), lambda i: (0, 0)),
            pl.BlockSpec((2 * d, 2 * d), lambda i: (0, 0)),
            pl.BlockSpec((1, 2 * d), lambda i: (0, 0)),
        ],
        out_specs=pl.BlockSpec((_BB, d), lambda i: (i, 0)),
        out_shape=jax.ShapeDtypeStruct((b_total, d), jnp.float32),
    )(g2, rt, r2e8, w1b, b1, w1blk, w2blk, b2p)


def kernel(nodes, history_uv, history_r, v2e_weight, u2e_weight, r2e_weight, W1, b1, W2, b2):
    b_total, ll = history_uv.shape
    _, d = v2e_weight.shape
    r = r2e_weight.shape[0]
    n = b_total * ll

    idx_flat = history_uv.reshape(-1).astype(jnp.int32)
    g_flat = _sc_gather(v2e_weight, idx_flat)
    g2 = g_flat.reshape(n // 2, 2 * d)

    rt = history_r.astype(jnp.int32).reshape(b_total // _BB, 1, _BB * ll)

    r2e8 = jnp.pad(r2e_weight, ((0, 8 - r), (0, 0)))
    w1a = W1[:d, :]
    w1b = W1[d:, :]
    zero = jnp.zeros((d, d), jnp.float32)
    w1blk = jnp.block([[w1a, zero], [zero, w1a]])
    w2blk = jnp.block([[W2, zero], [zero, W2]])
    b2p = jnp.concatenate([b2, b2]).reshape(1, 2 * d)

    return _tc_mlp(
        g2, rt, r2e8, w1b, b1.reshape(1, d),
        w1blk, w2blk, b2p, b_total=b_total, ll=ll, d=d, r=r,
    )
